# baseline (speedup 1.0000x reference)
import functools

import jax
import jax.numpy as jnp
from jax import lax
from jax.experimental import pallas as pl
from jax.experimental.pallas import tpu as pltpu
from jax.experimental.pallas import tpu_sc as plsc

N = 10000
NP = 10240
E = 320000
D_NODE = 128
TILES = 16
NW = 32
CH = 128
NCW_E = 80
NCW_C = 158
BR = 256
NBLK = NP // BR
NA = 10016
RPW = 640
RPL = NA - 15 * RPW
C_PRE = 128


def _make_sc_segsum(C, ncw):
    scratch = [
        pltpu.VMEM((4, CH), jnp.int32),
        pltpu.VMEM((5, CH), jnp.int32),
        pltpu.VMEM((3, CH, C), jnp.float32),
        pltpu.VMEM_SHARED((NA, C), jnp.float32),
        pltpu.SemaphoreType.DMA((4,)),
        pltpu.SemaphoreType.DMA((5,)),
        pltpu.SemaphoreType.DMA((3,)),
        pltpu.SemaphoreType.DMA((3,)),
    ]

    def body(tbl, gidx, sidx, z, out, gring, sring, bufs, agg,
             gsem, ssem, dsem, scsem):
        cid = lax.axis_index("c")
        sid = lax.axis_index("s")
        w = cid * TILES + sid

        def start_idx(j):
            sg, ss = lax.rem(j, 4), lax.rem(j, 5)
            pltpu.async_copy(gidx.at[w, j], gring.at[sg], gsem.at[sg])
            pltpu.async_copy(sidx.at[w, j], sring.at[ss], ssem.at[ss])

        def wait_idx(j):
            sg, ss = lax.rem(j, 4), lax.rem(j, 5)
            pltpu.make_async_copy(gidx.at[0, 0], gring.at[sg],
                                  gsem.at[sg]).wait()
            pltpu.make_async_copy(sidx.at[0, 0], sring.at[ss],
                                  ssem.at[ss]).wait()

        def start_gather(slot, b):
            pltpu.async_copy(tbl.at[gring.at[slot]], bufs.at[b], dsem.at[b])

        def wait_gather(b):
            pltpu.make_async_copy(tbl.at[gring.at[0]], bufs.at[b],
                                  dsem.at[b]).wait()

        def start_scatter(slot, b):
            pltpu.async_copy(bufs.at[b], agg.at[sring.at[slot]],
                             scsem.at[b], add=True)

        def wait_scatter(b):
            pltpu.make_async_copy(bufs.at[0], agg.at[sring.at[0]],
                                  scsem.at[b]).wait()

        @pl.when(sid < 15)
        def _():
            pltpu.sync_copy(z, agg.at[pl.ds(sid * RPW, RPW)])

        @pl.when(sid == 15)
        def _():
            pltpu.sync_copy(z.at[pl.ds(0, RPL)],
                            agg.at[pl.ds(15 * RPW, RPL)])

        for k in range(3):
            start_idx(k)
        wait_idx(0)
        start_gather(0, 0)
        plsc.subcore_barrier()

        def step(j, carry):
            @pl.when(j + 1 < ncw)
            def _():
                bn = lax.rem(j + 1, 3)

                @pl.when(j >= 2)
                def _():
                    wait_scatter(bn)

                wait_idx(j + 1)
                start_gather(lax.rem(j + 1, 4), bn)

            @pl.when(j + 3 < ncw)
            def _():
                start_idx(j + 3)

            b = lax.rem(j, 3)
            wait_gather(b)
            start_scatter(lax.rem(j, 5), b)
            return carry

        lax.fori_loop(0, ncw, step, 0)
        for b in range(3):
            wait_scatter(b)
        plsc.subcore_barrier()

        @pl.when(sid < 15)
        def _():
            pltpu.sync_copy(
                agg.at[pl.ds(sid * RPW, RPW)],
                out.at[pl.ds(cid * NP + sid * RPW, RPW)],
            )

        @pl.when(sid == 15)
        def _():
            pltpu.sync_copy(
                agg.at[pl.ds(15 * RPW, RPL)],
                out.at[pl.ds(cid * NP + 15 * RPW, RPL)],
            )

    return functools.partial(
        pl.kernel,
        out_type=jax.ShapeDtypeStruct((2 * NP, C), jnp.float32),
        mesh=plsc.VectorSubcoreMesh(core_axis_name="c", subcore_axis_name="s"),
        scratch_types=scratch,
    )(body)


def _row_spec(cols):
    return pl.BlockSpec((BR, cols), lambda r: (r, 0))


def _hi_spec(cols):
    return pl.BlockSpec((BR, cols), lambda r: (NBLK + r, 0))


def _full_spec(rows, cols):
    return pl.BlockSpec((rows, cols), lambda r: (0, 0))


def _tc_layer(fi, fo, first, prev_cols, out_split):
    def body(*refs):
        if first:
            (hin, pre0, pre1, wn, ws, we, b), outs = refs[:7], refs[7:]
        else:
            (hin, pa, pb, pre0, pre1, wn, ws, we, b), outs = refs[:9], refs[9:]
        pre = pre0[...] + pre1[...]
        inv = 1.0 / jnp.maximum(pre[:, :1], 1.0)
        amat = pre[:, 1:17]
        if first:
            h = hin[...]
        else:
            if prev_cols:
                pfull = jnp.concatenate([pa[...], pb[...]], axis=1)
            else:
                pfull = (pa[...] + pb[...])[:, :fi]
            h = jnp.maximum(hin[...] + inv * pfull, 0.0)
        u = (
            jnp.dot(h, ws[...], preferred_element_type=jnp.float32)
            + b[...]
            + inv * jnp.dot(amat, we[...], preferred_element_type=jnp.float32)
        )
        t = jnp.dot(h, wn[...], preferred_element_type=jnp.float32)
        if out_split:
            outs[0][...] = t[:, :128]
            outs[1][...] = t[:, 128:]
        else:
            outs[0][...] = t
            outs[1][...] = t
        outs[-1][...] = u

    wn_cols = 256 if out_split else 128
    in_specs = [_row_spec(fi)]
    if not first:
        in_specs += [_row_spec(128), _hi_spec(128)]
    in_specs += [
        _row_spec(C_PRE), _hi_spec(C_PRE),
        _full_spec(fi, wn_cols),
        _full_spec(fi, fo),
        _full_spec(16, fo),
        pl.BlockSpec((1, fo), lambda r: (0, 0)),
    ]
    t_shapes = [jax.ShapeDtypeStruct((NP, 128), jnp.float32)] * 2
    out_specs = [_row_spec(128)] * len(t_shapes) + [_row_spec(fo)]
    out_shape = t_shapes + [jax.ShapeDtypeStruct((NP, fo), jnp.float32)]
    return pl.pallas_call(
        body,
        grid=(NBLK,),
        in_specs=in_specs,
        out_specs=out_specs,
        out_shape=out_shape,
    )


def _tc_final():
    def body(u, pa, pb, pre0, pre1, out):
        pre = pre0[...] + pre1[...]
        inv = 1.0 / jnp.maximum(pre[:, :1], 1.0)
        out[...] = u[...] + inv * (pa[...][:, :1] + pb[...][:, :1])

    return pl.pallas_call(
        body,
        grid=(NBLK,),
        in_specs=[
            _row_spec(1), _row_spec(128), _hi_spec(128),
            _row_spec(C_PRE), _hi_spec(C_PRE),
        ],
        out_specs=_row_spec(1),
        out_shape=jax.ShapeDtypeStruct((NP, 1), jnp.float32),
    )


def _pad_rows(a, rows, val=0):
    return jnp.concatenate(
        [a, jnp.full((rows - a.shape[0],) + a.shape[1:], val, a.dtype)])


def kernel(x, edge_index, edge_attr, params):
    src = edge_index[0]
    dst = edge_index[1]

    e_pad = NW * NCW_E * CH
    src_e = _pad_rows(src, e_pad, N).reshape(NW, NCW_E, CH)
    dst_e = _pad_rows(dst, e_pad, N).reshape(NW, NCW_E, CH)
    gidx_e = jnp.concatenate([src_e[:TILES], src_e[TILES:] + NP], axis=0)

    c_pad = TILES * NCW_C * CH
    src_t = _pad_rows(src, c_pad, N).reshape(TILES, NCW_C, CH)
    dst_t = _pad_rows(dst, c_pad, N).reshape(TILES, NCW_C, CH)
    gidx_c = jnp.concatenate([src_t, src_t + NP], axis=0)
    sidx_c = jnp.concatenate([dst_t, dst_t], axis=0)

    x_pad = _pad_rows(x, NP).astype(jnp.float32)

    et = jnp.concatenate(
        [jnp.ones((E, 1), jnp.float32), edge_attr,
         jnp.zeros((E, C_PRE - 17), jnp.float32)], axis=1)
    tbl_pre = _pad_rows(et, e_pad)
    z_pre = jnp.zeros((RPW, C_PRE), jnp.float32)
    eid_e = jnp.arange(e_pad, dtype=jnp.int32).reshape(NW, NCW_E, CH)
    pre = _make_sc_segsum(C_PRE, NCW_E)(tbl_pre, eid_e, dst_e, z_pre)

    z128 = jnp.zeros((RPW, 128), jnp.float32)
    fo_list = [64, 128, 128, 256, 256, 256, 1]

    h_u = None
    p_prev = None
    prev_cols = False
    fi = D_NODE
    for i, p in enumerate(params):
        fo = fo_list[i]
        split = fo == 256
        wn_cols = 256 if split else 128
        wn = p['W_nbr']
        if wn.shape[1] != wn_cols:
            wn = jnp.zeros((fi, wn_cols), jnp.float32).at[:, :fo].set(wn)
        b = p['b'].reshape(1, fo)
        tc = _tc_layer(fi, fo, i == 0, prev_cols, split)
        if i == 0:
            outs = tc(x_pad, pre, pre, wn, p['W_self'], p['W_edge'], b)
        else:
            outs = tc(h_u, p_prev, p_prev, pre, pre, wn, p['W_self'],
                      p['W_edge'], b)
        h_u = outs[-1]
        tbl = jnp.concatenate([outs[0], outs[1]], axis=0)
        if split:
            p_prev = _make_sc_segsum(128, NCW_C)(tbl, gidx_c, sidx_c, z128)
        else:
            p_prev = _make_sc_segsum(128, NCW_E)(tbl, gidx_e, dst_e, z128)
        prev_cols = split
        fi = fo

    out = _tc_final()(h_u, p_prev, p_prev, pre, pre)
    return out[:N]


# speedup vs baseline: 1.0968x; 1.0968x over previous
import functools

import jax
import jax.numpy as jnp
from jax import lax
from jax.experimental import pallas as pl
from jax.experimental.pallas import tpu as pltpu
from jax.experimental.pallas import tpu_sc as plsc

N = 10000
NP = 10240
E = 320000
D_NODE = 128
TILES = 16
NW = 32
CH = 128
NCW_E = 80
NCW_C = 158
NBUF = 3
GD = 4
SD = 5
PF = 4
BR = 256
NBLK = NP // BR
NA = 10016
RPW = 640
RPL = NA - 15 * RPW
C_PRE = 128


def _make_sc_segsum(C, ncw):
    scratch = [
        pltpu.VMEM((GD, CH), jnp.int32),
        pltpu.VMEM((SD, CH), jnp.int32),
        pltpu.VMEM((NBUF, CH, C), jnp.float32),
        pltpu.VMEM_SHARED((NA, C), jnp.float32),
        pltpu.SemaphoreType.DMA((GD,)),
        pltpu.SemaphoreType.DMA((SD,)),
        pltpu.SemaphoreType.DMA((NBUF,)),
        pltpu.SemaphoreType.DMA((NBUF,)),
    ]

    def body(tbl, gidx, sidx, z, out, gring, sring, bufs, agg,
             gsem, ssem, dsem, scsem):
        cid = lax.axis_index("c")
        sid = lax.axis_index("s")
        w = cid * TILES + sid

        def start_idx(j):
            sg, ss = lax.rem(j, GD), lax.rem(j, SD)
            pltpu.async_copy(gidx.at[w, j], gring.at[sg], gsem.at[sg])
            pltpu.async_copy(sidx.at[w, j], sring.at[ss], ssem.at[ss])

        def wait_idx(j):
            sg, ss = lax.rem(j, GD), lax.rem(j, SD)
            pltpu.make_async_copy(gidx.at[0, 0], gring.at[sg],
                                  gsem.at[sg]).wait()
            pltpu.make_async_copy(sidx.at[0, 0], sring.at[ss],
                                  ssem.at[ss]).wait()

        def start_gather(slot, b):
            pltpu.async_copy(tbl.at[gring.at[slot]], bufs.at[b], dsem.at[b])

        def wait_gather(b):
            pltpu.make_async_copy(tbl.at[gring.at[0]], bufs.at[b],
                                  dsem.at[b]).wait()

        def start_scatter(slot, b):
            pltpu.async_copy(bufs.at[b], agg.at[sring.at[slot]],
                             scsem.at[b], add=True)

        def wait_scatter(b):
            pltpu.make_async_copy(bufs.at[0], agg.at[sring.at[0]],
                                  scsem.at[b]).wait()

        @pl.when(sid < 15)
        def _():
            pltpu.sync_copy(z, agg.at[pl.ds(sid * RPW, RPW)])

        @pl.when(sid == 15)
        def _():
            pltpu.sync_copy(z.at[pl.ds(0, RPL)],
                            agg.at[pl.ds(15 * RPW, RPL)])

        for k in range(PF):
            start_idx(k)
        for k in range(2):
            wait_idx(k)
            start_gather(k, k)
        plsc.subcore_barrier()

        def step(j, carry):
            @pl.when(j + 2 < ncw)
            def _():
                bn = lax.rem(j + 2, NBUF)

                @pl.when(j >= 1)
                def _():
                    wait_scatter(bn)

                wait_idx(j + 2)
                start_gather(lax.rem(j + 2, GD), bn)

            b = lax.rem(j, NBUF)
            wait_gather(b)

            @pl.when(j + PF < ncw)
            def _():
                start_idx(j + PF)

            start_scatter(lax.rem(j, SD), b)
            return carry

        lax.fori_loop(0, ncw, step, 0)
        for b in range(NBUF):
            wait_scatter(b)
        plsc.subcore_barrier()

        @pl.when(sid < 15)
        def _():
            pltpu.sync_copy(
                agg.at[pl.ds(sid * RPW, RPW)],
                out.at[pl.ds(cid * NP + sid * RPW, RPW)],
            )

        @pl.when(sid == 15)
        def _():
            pltpu.sync_copy(
                agg.at[pl.ds(15 * RPW, RPL)],
                out.at[pl.ds(cid * NP + 15 * RPW, RPL)],
            )

    return functools.partial(
        pl.kernel,
        out_type=jax.ShapeDtypeStruct((2 * NP, C), jnp.float32),
        mesh=plsc.VectorSubcoreMesh(core_axis_name="c", subcore_axis_name="s"),
        scratch_types=scratch,
    )(body)


def _row_spec(cols):
    return pl.BlockSpec((BR, cols), lambda r: (r, 0))


def _hi_spec(cols):
    return pl.BlockSpec((BR, cols), lambda r: (NBLK + r, 0))


def _full_spec(rows, cols):
    return pl.BlockSpec((rows, cols), lambda r: (0, 0))


def _tc_layer(fi, fo, first, prev_cols, out_split):
    def body(*refs):
        if first:
            (hin, pre0, pre1, wn, ws, we, b), outs = refs[:7], refs[7:]
        else:
            (hin, pa, pb, pre0, pre1, wn, ws, we, b), outs = refs[:9], refs[9:]
        pre = pre0[...] + pre1[...]
        inv = 1.0 / jnp.maximum(pre[:, :1], 1.0)
        amat = pre[:, 1:17]
        if first:
            h = hin[...]
        else:
            if prev_cols:
                pfull = jnp.concatenate([pa[...], pb[...]], axis=1)
            else:
                pfull = (pa[...] + pb[...])[:, :fi]
            h = jnp.maximum(hin[...] + inv * pfull, 0.0)
        u = (
            jnp.dot(h, ws[...], preferred_element_type=jnp.float32)
            + b[...]
            + inv * jnp.dot(amat, we[...], preferred_element_type=jnp.float32)
        )
        t = jnp.dot(h, wn[...], preferred_element_type=jnp.float32)
        if out_split:
            outs[0][...] = t[:, :128]
            outs[1][...] = t[:, 128:]
        else:
            outs[0][...] = t
        outs[-1][...] = u

    wn_cols = 256 if out_split else 128
    in_specs = [_row_spec(fi)]
    if not first:
        in_specs += [_row_spec(128), _hi_spec(128)]
    in_specs += [
        _row_spec(C_PRE), _hi_spec(C_PRE),
        _full_spec(fi, wn_cols),
        _full_spec(fi, fo),
        _full_spec(16, fo),
        pl.BlockSpec((1, fo), lambda r: (0, 0)),
    ]
    t_shapes = (
        [jax.ShapeDtypeStruct((NP, 128), jnp.float32)] * 2 if out_split
        else [jax.ShapeDtypeStruct((NP, 128), jnp.float32)]
    )
    out_specs = [_row_spec(128)] * len(t_shapes) + [_row_spec(fo)]
    out_shape = t_shapes + [jax.ShapeDtypeStruct((NP, fo), jnp.float32)]
    return pl.pallas_call(
        body,
        grid=(NBLK,),
        in_specs=in_specs,
        out_specs=out_specs,
        out_shape=out_shape,
    )


def _tc_final():
    def body(u, pa, pb, pre0, pre1, out):
        pre = pre0[...] + pre1[...]
        inv = 1.0 / jnp.maximum(pre[:, :1], 1.0)
        out[...] = u[...] + inv * (pa[...][:, :1] + pb[...][:, :1])

    return pl.pallas_call(
        body,
        grid=(NBLK,),
        in_specs=[
            _row_spec(1), _row_spec(128), _hi_spec(128),
            _row_spec(C_PRE), _hi_spec(C_PRE),
        ],
        out_specs=_row_spec(1),
        out_shape=jax.ShapeDtypeStruct((NP, 1), jnp.float32),
    )


def _pad_rows(a, rows, val=0):
    return jnp.concatenate(
        [a, jnp.full((rows - a.shape[0],) + a.shape[1:], val, a.dtype)])


def kernel(x, edge_index, edge_attr, params):
    src = edge_index[0]
    dst = edge_index[1]

    e_pad = NW * NCW_E * CH
    src_e = _pad_rows(src, e_pad, N).reshape(NW, NCW_E, CH)
    dst_e = _pad_rows(dst, e_pad, N).reshape(NW, NCW_E, CH)

    c_pad = TILES * NCW_C * CH
    src_t = _pad_rows(src, c_pad, N).reshape(TILES, NCW_C, CH)
    dst_t = _pad_rows(dst, c_pad, N).reshape(TILES, NCW_C, CH)
    gidx_c = jnp.concatenate([src_t, src_t + NP], axis=0)
    sidx_c = jnp.concatenate([dst_t, dst_t], axis=0)

    x_pad = _pad_rows(x, NP).astype(jnp.float32)

    et = jnp.concatenate(
        [jnp.ones((E, 1), jnp.float32), edge_attr,
         jnp.zeros((E, C_PRE - 17), jnp.float32)], axis=1)
    tbl_pre = _pad_rows(et, e_pad)
    z_pre = jnp.zeros((RPW, C_PRE), jnp.float32)
    eid_e = jnp.arange(e_pad, dtype=jnp.int32).reshape(NW, NCW_E, CH)
    pre = _make_sc_segsum(C_PRE, NCW_E)(tbl_pre, eid_e, dst_e, z_pre)

    z128 = jnp.zeros((RPW, 128), jnp.float32)
    fo_list = [64, 128, 128, 256, 256, 256, 1]

    h_u = None
    p_prev = None
    prev_cols = False
    fi = D_NODE
    for i, p in enumerate(params):
        fo = fo_list[i]
        split = fo == 256
        wn_cols = 256 if split else 128
        wn = p['W_nbr']
        if wn.shape[1] != wn_cols:
            wn = jnp.zeros((fi, wn_cols), jnp.float32).at[:, :fo].set(wn)
        b = p['b'].reshape(1, fo)
        tc = _tc_layer(fi, fo, i == 0, prev_cols, split)
        if i == 0:
            outs = tc(x_pad, pre, pre, wn, p['W_self'], p['W_edge'], b)
        else:
            outs = tc(h_u, p_prev, p_prev, pre, pre, wn, p['W_self'],
                      p['W_edge'], b)
        h_u = outs[-1]
        if split:
            tbl = jnp.concatenate([outs[0], outs[1]], axis=0)
            p_prev = _make_sc_segsum(128, NCW_C)(tbl, gidx_c, sidx_c, z128)
        else:
            p_prev = _make_sc_segsum(128, NCW_E)(outs[0], src_e, dst_e, z128)
        prev_cols = split
        fi = fo

    out = _tc_final()(h_u, p_prev, p_prev, pre, pre)
    return out[:N]
